# baseline (device time: 159223 ns/iter reference)
import jax
import jax.numpy as jnp
from jax import lax
from jax.experimental import pallas as pl
from jax.experimental.pallas import tpu as pltpu

N_DEV = 4


def kernel(x, k, Wp):
    B, S, C = x.shape
    T = k.shape[0]
    N = Wp.shape[1]

    def body(x_ref, k_ref, w_ref, out_ref, comm_ref, send_sems, recv_sems):
        my = lax.axis_index("i")
        left = lax.rem(my + (N_DEV - 1), N_DEV)
        right = lax.rem(my + 1, N_DEV)

        barrier = pltpu.get_barrier_semaphore()
        for nbr in (left, right):
            pl.semaphore_signal(
                barrier, inc=1,
                device_id=(nbr,), device_id_type=pl.DeviceIdType.MESH,
            )
        pl.semaphore_wait(barrier, 2)

        xv = x_ref[...]
        kv = k_ref[...]
        y = xv * kv[T - 1]
        for d in range(1, T):
            shifted = jnp.concatenate(
                [jnp.zeros((B, d, C), xv.dtype), xv[:, : S - d, :]], axis=1
            )
            y = y + shifted * kv[T - 1 - d]
        a = y * jax.nn.sigmoid(y)
        a2 = a.reshape(B * S, C).astype(jnp.bfloat16)
        w = w_ref[...].astype(jnp.bfloat16)
        part = jnp.dot(a2, w, preferred_element_type=jnp.float32)
        part = part.reshape(B, S, N)
        out_ref[...] = part
        comm_ref[0] = part.astype(jnp.bfloat16)

        for h in range(N_DEV - 1):
            rdma = pltpu.make_async_remote_copy(
                src_ref=comm_ref.at[h],
                dst_ref=comm_ref.at[h + 1],
                send_sem=send_sems.at[h],
                recv_sem=recv_sems.at[h + 1],
                device_id=(right,),
                device_id_type=pl.DeviceIdType.MESH,
            )
            rdma.start()
            rdma.wait()
            out_ref[...] = out_ref[...] + comm_ref[h + 1].astype(jnp.float32)

    return pl.pallas_call(
        body,
        out_shape=jax.ShapeDtypeStruct((B, S, N), jnp.float32),
        in_specs=[
            pl.BlockSpec(memory_space=pltpu.VMEM),
            pl.BlockSpec(memory_space=pltpu.VMEM),
            pl.BlockSpec(memory_space=pltpu.VMEM),
        ],
        out_specs=pl.BlockSpec(memory_space=pltpu.VMEM),
        scratch_shapes=[
            pltpu.VMEM((N_DEV, B, S, N), jnp.bfloat16),
            pltpu.SemaphoreType.DMA((N_DEV,)),
            pltpu.SemaphoreType.DMA((N_DEV,)),
        ],
        compiler_params=pltpu.CompilerParams(collective_id=0),
    )(x, k, Wp)


# device time: 67685 ns/iter; 2.3524x vs baseline; 2.3524x over previous
import jax
import jax.numpy as jnp
from jax import lax
from jax.experimental import pallas as pl
from jax.experimental.pallas import tpu as pltpu

N_DEV = 4


def kernel(x, k, Wp):
    B, S, C = x.shape
    T = k.shape[0]
    N = Wp.shape[1]

    def body(x_ref, k_ref, w_ref, out_ref,
             part_bf, rs_recv, ag_send, ag_recv,
             rs_send_sems, rs_recv_sems, ag_send_sems, ag_recv_sems):
        my = lax.axis_index("i")

        barrier = pltpu.get_barrier_semaphore()
        for j in range(N_DEV - 1):
            peer = lax.rem(my + (j + 1), N_DEV)
            pl.semaphore_signal(
                barrier, inc=1,
                device_id=(peer,), device_id_type=pl.DeviceIdType.MESH,
            )
        pl.semaphore_wait(barrier, N_DEV - 1)

        xv = x_ref[...]
        kv = k_ref[...]
        y = xv * kv[T - 1]
        for d in range(1, T):
            shifted = jnp.concatenate(
                [jnp.zeros((B, d, C), xv.dtype), xv[:, : S - d, :]], axis=1
            )
            y = y + shifted * kv[T - 1 - d]
        a = y * jax.nn.sigmoid(y)
        a2 = a.reshape(B * S, C).astype(jnp.bfloat16)
        w = w_ref[...].astype(jnp.bfloat16)
        part = jnp.dot(a2, w, preferred_element_type=jnp.float32)
        part = part.reshape(B, S, N)
        out_ref[...] = part
        part_bf[...] = part.astype(jnp.bfloat16)

        rs = []
        for j in range(N_DEV - 1):
            peer = lax.rem(my + (j + 1), N_DEV)
            rdma = pltpu.make_async_remote_copy(
                src_ref=part_bf.at[pl.ds(peer, 1)],
                dst_ref=rs_recv.at[pl.ds(N_DEV - 2 - j, 1)],
                send_sem=rs_send_sems.at[j],
                recv_sem=rs_recv_sems.at[N_DEV - 2 - j],
                device_id=(peer,),
                device_id_type=pl.DeviceIdType.MESH,
            )
            rdma.start()
            rs.append(rdma)
        for rdma in rs:
            rdma.wait()

        red = out_ref[pl.ds(my, 1)]
        for j in range(N_DEV - 1):
            red = red + rs_recv[pl.ds(j, 1)].astype(jnp.float32)
        out_ref[pl.ds(my, 1)] = red
        ag_send[...] = red.astype(jnp.bfloat16)

        ag = []
        for j in range(N_DEV - 1):
            peer = lax.rem(my + (j + 1), N_DEV)
            rdma = pltpu.make_async_remote_copy(
                src_ref=ag_send,
                dst_ref=ag_recv.at[pl.ds(N_DEV - 2 - j, 1)],
                send_sem=ag_send_sems.at[j],
                recv_sem=ag_recv_sems.at[N_DEV - 2 - j],
                device_id=(peer,),
                device_id_type=pl.DeviceIdType.MESH,
            )
            rdma.start()
            ag.append(rdma)
        for j in range(N_DEV - 1):
            ag[j].wait()
            origin = lax.rem(my + (N_DEV - 1 - j), N_DEV)
            out_ref[pl.ds(origin, 1)] = (
                ag_recv[pl.ds(N_DEV - 2 - j, 1)].astype(jnp.float32)
            )

    return pl.pallas_call(
        body,
        out_shape=jax.ShapeDtypeStruct((B, S, N), jnp.float32),
        in_specs=[
            pl.BlockSpec(memory_space=pltpu.VMEM),
            pl.BlockSpec(memory_space=pltpu.VMEM),
            pl.BlockSpec(memory_space=pltpu.VMEM),
        ],
        out_specs=pl.BlockSpec(memory_space=pltpu.VMEM),
        scratch_shapes=[
            pltpu.VMEM((B, S, N), jnp.bfloat16),
            pltpu.VMEM((N_DEV - 1, S, N), jnp.bfloat16),
            pltpu.VMEM((1, S, N), jnp.bfloat16),
            pltpu.VMEM((N_DEV - 1, S, N), jnp.bfloat16),
            pltpu.SemaphoreType.DMA((N_DEV - 1,)),
            pltpu.SemaphoreType.DMA((N_DEV - 1,)),
            pltpu.SemaphoreType.DMA((N_DEV - 1,)),
            pltpu.SemaphoreType.DMA((N_DEV - 1,)),
        ],
        compiler_params=pltpu.CompilerParams(collective_id=0),
    )(x, k, Wp)


# device time: 62677 ns/iter; 2.5404x vs baseline; 1.0799x over previous
import jax
import jax.numpy as jnp
from jax import lax
from jax.experimental import pallas as pl
from jax.experimental.pallas import tpu as pltpu

N_DEV = 4


def kernel(x, k, Wp):
    B, S, C = x.shape
    T = k.shape[0]
    N = Wp.shape[1]

    def body(x_ref, k_ref, w_ref, out_ref,
             part_bf, rs_recv, ag_send, ag_recv,
             rs_send_sems, rs_recv_sems, ag_send_sems, ag_recv_sems):
        my = lax.axis_index("i")

        barrier = pltpu.get_barrier_semaphore()
        for j in range(N_DEV - 1):
            peer = lax.rem(my + (j + 1), N_DEV)
            pl.semaphore_signal(
                barrier, inc=1,
                device_id=(peer,), device_id_type=pl.DeviceIdType.MESH,
            )
        pl.semaphore_wait(barrier, N_DEV - 1)

        kb = k_ref[...].astype(jnp.bfloat16)
        w = w_ref[...].astype(jnp.bfloat16)

        def compute_chunk(q):
            xq = x_ref[pl.ds(q, 1)].astype(jnp.bfloat16)
            y = xq * kb[T - 1]
            for d in range(1, T):
                shifted = jnp.concatenate(
                    [jnp.zeros((1, d, C), jnp.bfloat16), xq[:, : S - d, :]],
                    axis=1,
                )
                y = y + shifted * kb[T - 1 - d]
            a = y * jax.nn.sigmoid(y)
            return jnp.dot(
                a.reshape(S, C), w, preferred_element_type=jnp.float32
            ).reshape(1, S, N)

        rs = []
        for j in range(N_DEV - 1):
            peer = lax.rem(my + (j + 1), N_DEV)
            part_bf[pl.ds(peer, 1)] = compute_chunk(peer).astype(jnp.bfloat16)
            rdma = pltpu.make_async_remote_copy(
                src_ref=part_bf.at[pl.ds(peer, 1)],
                dst_ref=rs_recv.at[pl.ds(N_DEV - 2 - j, 1)],
                send_sem=rs_send_sems.at[j],
                recv_sem=rs_recv_sems.at[N_DEV - 2 - j],
                device_id=(peer,),
                device_id_type=pl.DeviceIdType.MESH,
            )
            rdma.start()
            rs.append(rdma)

        red = compute_chunk(my)
        for j in range(N_DEV - 1):
            rs[j].wait()
            red = red + rs_recv[pl.ds(N_DEV - 2 - j, 1)].astype(jnp.float32)
        out_ref[pl.ds(my, 1)] = red
        ag_send[...] = red.astype(jnp.bfloat16)

        ag = []
        for j in range(N_DEV - 1):
            peer = lax.rem(my + (j + 1), N_DEV)
            rdma = pltpu.make_async_remote_copy(
                src_ref=ag_send,
                dst_ref=ag_recv.at[pl.ds(N_DEV - 2 - j, 1)],
                send_sem=ag_send_sems.at[j],
                recv_sem=ag_recv_sems.at[N_DEV - 2 - j],
                device_id=(peer,),
                device_id_type=pl.DeviceIdType.MESH,
            )
            rdma.start()
            ag.append(rdma)
        for j in range(N_DEV - 1):
            ag[j].wait()
            origin = lax.rem(my + (N_DEV - 1 - j), N_DEV)
            out_ref[pl.ds(origin, 1)] = (
                ag_recv[pl.ds(N_DEV - 2 - j, 1)].astype(jnp.float32)
            )

    return pl.pallas_call(
        body,
        out_shape=jax.ShapeDtypeStruct((B, S, N), jnp.float32),
        in_specs=[
            pl.BlockSpec(memory_space=pltpu.VMEM),
            pl.BlockSpec(memory_space=pltpu.VMEM),
            pl.BlockSpec(memory_space=pltpu.VMEM),
        ],
        out_specs=pl.BlockSpec(memory_space=pltpu.VMEM),
        scratch_shapes=[
            pltpu.VMEM((B, S, N), jnp.bfloat16),
            pltpu.VMEM((N_DEV - 1, S, N), jnp.bfloat16),
            pltpu.VMEM((1, S, N), jnp.bfloat16),
            pltpu.VMEM((N_DEV - 1, S, N), jnp.bfloat16),
            pltpu.SemaphoreType.DMA((N_DEV - 1,)),
            pltpu.SemaphoreType.DMA((N_DEV - 1,)),
            pltpu.SemaphoreType.DMA((N_DEV - 1,)),
            pltpu.SemaphoreType.DMA((N_DEV - 1,)),
        ],
        compiler_params=pltpu.CompilerParams(collective_id=0),
    )(x, k, Wp)


# device time: 61176 ns/iter; 2.6027x vs baseline; 1.0245x over previous
import jax
import jax.numpy as jnp
from jax import lax
from jax.experimental import pallas as pl
from jax.experimental.pallas import tpu as pltpu

N_DEV = 4


def kernel(x, k, Wp):
    B, S, C = x.shape
    T = k.shape[0]
    N = Wp.shape[1]

    def body(x_ref, k_ref, w_ref, out_ref,
             part_bf, rs_recv, ag_send, ag_recv,
             rs_send_sems, rs_recv_sems, ag_send_sems, ag_recv_sems):
        my = lax.axis_index("i")

        barrier = pltpu.get_barrier_semaphore()
        for j in range(N_DEV - 1):
            peer = lax.rem(my + (j + 1), N_DEV)
            pl.semaphore_signal(
                barrier, inc=1,
                device_id=(peer,), device_id_type=pl.DeviceIdType.MESH,
            )
        pl.semaphore_wait(barrier, N_DEV - 1)

        kb = k_ref[...].astype(jnp.bfloat16)
        w = w_ref[...].astype(jnp.bfloat16)

        def compute_chunk(q):
            xq = x_ref[pl.ds(q, 1)].astype(jnp.bfloat16)
            y = xq * kb[T - 1]
            for d in range(1, T):
                shifted = jnp.concatenate(
                    [jnp.zeros((1, d, C), jnp.bfloat16), xq[:, : S - d, :]],
                    axis=1,
                )
                y = y + shifted * kb[T - 1 - d]
            a = y * jax.nn.sigmoid(y)
            return jnp.dot(
                a.reshape(S, C), w, preferred_element_type=jnp.float32
            ).reshape(1, S, N)

        SUB = 2
        S2 = S // SUB
        rs = []
        for j in range(N_DEV - 1):
            peer = lax.rem(my + (j + 1), N_DEV)
            part_bf[pl.ds(peer, 1)] = compute_chunk(peer).astype(jnp.bfloat16)
            subs = []
            for u in range(SUB):
                rdma = pltpu.make_async_remote_copy(
                    src_ref=part_bf.at[pl.ds(peer, 1), pl.ds(u * S2, S2)],
                    dst_ref=rs_recv.at[pl.ds(N_DEV - 2 - j, 1),
                                       pl.ds(u * S2, S2)],
                    send_sem=rs_send_sems.at[j, u],
                    recv_sem=rs_recv_sems.at[N_DEV - 2 - j, u],
                    device_id=(peer,),
                    device_id_type=pl.DeviceIdType.MESH,
                )
                rdma.start()
                subs.append(rdma)
            rs.append(subs)

        own = compute_chunk(my)
        ag = []
        for u in range(SUB):
            red_u = own[:, u * S2:(u + 1) * S2, :]
            for j in range(N_DEV - 1):
                rs[j][u].wait()
            for j in range(N_DEV - 1):
                red_u = red_u + rs_recv[
                    pl.ds(N_DEV - 2 - j, 1), pl.ds(u * S2, S2)
                ].astype(jnp.float32)
            out_ref[pl.ds(my, 1), pl.ds(u * S2, S2)] = red_u
            ag_send[pl.ds(0, 1), pl.ds(u * S2, S2)] = red_u.astype(jnp.bfloat16)
            for j in range(N_DEV - 1):
                peer = lax.rem(my + (j + 1), N_DEV)
                rdma = pltpu.make_async_remote_copy(
                    src_ref=ag_send.at[pl.ds(0, 1), pl.ds(u * S2, S2)],
                    dst_ref=ag_recv.at[pl.ds(N_DEV - 2 - j, 1),
                                       pl.ds(u * S2, S2)],
                    send_sem=ag_send_sems.at[j, u],
                    recv_sem=ag_recv_sems.at[N_DEV - 2 - j, u],
                    device_id=(peer,),
                    device_id_type=pl.DeviceIdType.MESH,
                )
                rdma.start()
                ag.append((rdma, j, u))

        for rdma, j, u in ag:
            rdma.wait()
            origin = lax.rem(my + (N_DEV - 1 - j), N_DEV)
            out_ref[pl.ds(origin, 1), pl.ds(u * S2, S2)] = ag_recv[
                pl.ds(N_DEV - 2 - j, 1), pl.ds(u * S2, S2)
            ].astype(jnp.float32)

    return pl.pallas_call(
        body,
        out_shape=jax.ShapeDtypeStruct((B, S, N), jnp.float32),
        in_specs=[
            pl.BlockSpec(memory_space=pltpu.VMEM),
            pl.BlockSpec(memory_space=pltpu.VMEM),
            pl.BlockSpec(memory_space=pltpu.VMEM),
        ],
        out_specs=pl.BlockSpec(memory_space=pltpu.VMEM),
        scratch_shapes=[
            pltpu.VMEM((B, S, N), jnp.bfloat16),
            pltpu.VMEM((N_DEV - 1, S, N), jnp.bfloat16),
            pltpu.VMEM((1, S, N), jnp.bfloat16),
            pltpu.VMEM((N_DEV - 1, S, N), jnp.bfloat16),
            pltpu.SemaphoreType.DMA((N_DEV - 1, 2)),
            pltpu.SemaphoreType.DMA((N_DEV - 1, 2)),
            pltpu.SemaphoreType.DMA((N_DEV - 1, 2)),
            pltpu.SemaphoreType.DMA((N_DEV - 1, 2)),
        ],
        compiler_params=pltpu.CompilerParams(collective_id=0),
    )(x, k, Wp)


# device time: 60097 ns/iter; 2.6494x vs baseline; 1.0180x over previous
import jax
import jax.numpy as jnp
from jax import lax
from jax.experimental import pallas as pl
from jax.experimental.pallas import tpu as pltpu

N_DEV = 4
SUB = 4
HALO = 3


def kernel(x, k, Wp):
    B, S, C = x.shape
    T = k.shape[0]
    N = Wp.shape[1]
    S2 = S // SUB

    def body(x_ref, k_ref, w_ref, out_ref,
             part_bf, rs_recv, ag_send, ag_recv,
             rs_send_sems, rs_recv_sems, ag_send_sems, ag_recv_sems):
        my = lax.axis_index("i")

        barrier = pltpu.get_barrier_semaphore()
        for j in range(N_DEV - 1):
            peer = lax.rem(my + (j + 1), N_DEV)
            pl.semaphore_signal(
                barrier, inc=1,
                device_id=(peer,), device_id_type=pl.DeviceIdType.MESH,
            )
        pl.semaphore_wait(barrier, N_DEV - 1)

        kb = k_ref[...].astype(jnp.bfloat16)
        w = w_ref[...].astype(jnp.bfloat16)

        def compute_sub(q, u):
            if u == 0:
                xq = x_ref[pl.ds(q, 1), pl.ds(0, S2)].astype(jnp.bfloat16)
                xp = jnp.concatenate(
                    [jnp.zeros((1, HALO, C), jnp.bfloat16), xq], axis=1
                )
            else:
                xp = x_ref[pl.ds(q, 1), pl.ds(u * S2 - HALO, S2 + HALO)]
                xp = xp.astype(jnp.bfloat16)
            y = xp[:, HALO:HALO + S2, :] * kb[T - 1]
            for t in range(T - 1):
                y = y + xp[:, t:t + S2, :] * kb[t]
            a = y * jax.nn.sigmoid(y)
            return jnp.dot(
                a.reshape(S2, C), w, preferred_element_type=jnp.float32
            ).reshape(1, S2, N)

        rs = [[None] * SUB for _ in range(N_DEV - 1)]
        own = []
        for u in range(SUB):
            for j in range(N_DEV - 1):
                peer = lax.rem(my + (j + 1), N_DEV)
                part_bf[pl.ds(peer, 1), pl.ds(u * S2, S2)] = (
                    compute_sub(peer, u).astype(jnp.bfloat16)
                )
                rdma = pltpu.make_async_remote_copy(
                    src_ref=part_bf.at[pl.ds(peer, 1), pl.ds(u * S2, S2)],
                    dst_ref=rs_recv.at[pl.ds(N_DEV - 2 - j, 1),
                                       pl.ds(u * S2, S2)],
                    send_sem=rs_send_sems.at[j, u],
                    recv_sem=rs_recv_sems.at[N_DEV - 2 - j, u],
                    device_id=(peer,),
                    device_id_type=pl.DeviceIdType.MESH,
                )
                rdma.start()
                rs[j][u] = rdma
            own.append(compute_sub(my, u))

        ag = []
        for u in range(SUB):
            for j in range(N_DEV - 1):
                rs[j][u].wait()
            red = own[u]
            for j in range(N_DEV - 1):
                red = red + rs_recv[
                    pl.ds(N_DEV - 2 - j, 1), pl.ds(u * S2, S2)
                ].astype(jnp.float32)
            out_ref[pl.ds(my, 1), pl.ds(u * S2, S2)] = red
            ag_send[pl.ds(0, 1), pl.ds(u * S2, S2)] = red.astype(jnp.bfloat16)
            for j in range(N_DEV - 1):
                peer = lax.rem(my + (j + 1), N_DEV)
                rdma = pltpu.make_async_remote_copy(
                    src_ref=ag_send.at[pl.ds(0, 1), pl.ds(u * S2, S2)],
                    dst_ref=ag_recv.at[pl.ds(N_DEV - 2 - j, 1),
                                       pl.ds(u * S2, S2)],
                    send_sem=ag_send_sems.at[j, u],
                    recv_sem=ag_recv_sems.at[N_DEV - 2 - j, u],
                    device_id=(peer,),
                    device_id_type=pl.DeviceIdType.MESH,
                )
                rdma.start()
                ag.append((rdma, j, u))

        for rdma, j, u in ag:
            rdma.wait()
            origin = lax.rem(my + (N_DEV - 1 - j), N_DEV)
            out_ref[pl.ds(origin, 1), pl.ds(u * S2, S2)] = ag_recv[
                pl.ds(N_DEV - 2 - j, 1), pl.ds(u * S2, S2)
            ].astype(jnp.float32)

    return pl.pallas_call(
        body,
        out_shape=jax.ShapeDtypeStruct((B, S, N), jnp.float32),
        in_specs=[
            pl.BlockSpec(memory_space=pltpu.VMEM),
            pl.BlockSpec(memory_space=pltpu.VMEM),
            pl.BlockSpec(memory_space=pltpu.VMEM),
        ],
        out_specs=pl.BlockSpec(memory_space=pltpu.VMEM),
        scratch_shapes=[
            pltpu.VMEM((B, S, N), jnp.bfloat16),
            pltpu.VMEM((N_DEV - 1, S, N), jnp.bfloat16),
            pltpu.VMEM((1, S, N), jnp.bfloat16),
            pltpu.VMEM((N_DEV - 1, S, N), jnp.bfloat16),
            pltpu.SemaphoreType.DMA((N_DEV - 1, SUB)),
            pltpu.SemaphoreType.DMA((N_DEV - 1, SUB)),
            pltpu.SemaphoreType.DMA((N_DEV - 1, SUB)),
            pltpu.SemaphoreType.DMA((N_DEV - 1, SUB)),
        ],
        compiler_params=pltpu.CompilerParams(collective_id=0),
    )(x, k, Wp)


# device time: 54921 ns/iter; 2.8991x vs baseline; 1.0942x over previous
import jax
import jax.numpy as jnp
from jax import lax
from jax.experimental import pallas as pl
from jax.experimental.pallas import tpu as pltpu

N_DEV = 4
N_SEM = 11


def kernel(x, k, Wp):
    B, S, C = x.shape
    T = k.shape[0]
    N = Wp.shape[1]
    H = S // 2

    def body(x_ref, k_ref, w_ref, out_ref, partbuf, rbuf,
             send_sems, recv_sems):
        my = lax.axis_index("i")
        r = lax.rem(my, 2)
        xp = 3 - my
        yp = my + 1 - 2 * r
        kps = my - r
        sps = 2 - kps
        srow2 = 2 - my + 2 * r

        barrier = pltpu.get_barrier_semaphore()
        for nbr in (xp, yp):
            pl.semaphore_signal(
                barrier, inc=1,
                device_id=(nbr,), device_id_type=pl.DeviceIdType.MESH,
            )
        pl.semaphore_wait(barrier, 2)

        kb = k_ref[...].astype(jnp.bfloat16)
        w = w_ref[...].astype(jnp.bfloat16)

        def compute_chunk(q):
            xq = x_ref[pl.ds(q, 1)].astype(jnp.bfloat16)
            xpad = jnp.concatenate(
                [jnp.zeros((1, T - 1, C), jnp.bfloat16), xq], axis=1
            )
            y = xpad[:, T - 1:T - 1 + S, :] * kb[T - 1]
            for t in range(T - 1):
                y = y + xpad[:, t:t + S, :] * kb[t]
            a = y * jax.nn.sigmoid(y)
            return jnp.dot(
                a.reshape(S, C), w, preferred_element_type=jnp.float32
            ).reshape(1, S, N).astype(jnp.bfloat16)

        def copy(i, src, dst, dev):
            return pltpu.make_async_remote_copy(
                src_ref=src, dst_ref=dst,
                send_sem=send_sems.at[i], recv_sem=recv_sems.at[i],
                device_id=(dev,), device_id_type=pl.DeviceIdType.MESH,
            )

        A = pl.ds(0, H)
        Bt = pl.ds(H, H)

        partbuf[pl.ds(srow2, 1)] = compute_chunk(srow2)
        rs1b2 = copy(2, partbuf.at[pl.ds(srow2, 1), Bt],
                     rbuf.at[pl.ds(3, 1)], yp)
        rs1b2.start()
        partbuf[pl.ds(3 - my, 1)] = compute_chunk(3 - my)
        rs1a = copy(0, partbuf.at[pl.ds(sps, 2), A],
                    rbuf.at[pl.ds(0, 2)], xp)
        rs1a.start()
        partbuf[pl.ds(yp, 1)] = compute_chunk(yp)
        rs1b1 = copy(1, partbuf.at[pl.ds(yp, 1), Bt],
                     rbuf.at[pl.ds(2, 1)], yp)
        rs1b1.start()
        partbuf[pl.ds(my, 1)] = compute_chunk(my)

        rs1a.wait()
        partbuf[pl.ds(kps, 2), A] = (
            partbuf[pl.ds(kps, 2), A] + rbuf[pl.ds(0, 2)]
        )
        rs1b1.wait()
        partbuf[pl.ds(my, 1), Bt] = (
            partbuf[pl.ds(my, 1), Bt] + rbuf[pl.ds(2, 1)]
        )
        rs1b2.wait()
        partbuf[pl.ds(3 - my, 1), Bt] = (
            partbuf[pl.ds(3 - my, 1), Bt] + rbuf[pl.ds(3, 1)]
        )

        rs2a = copy(3, partbuf.at[pl.ds(yp, 1), A],
                    rbuf.at[pl.ds(4, 1)], yp)
        rs2b = copy(4, partbuf.at[pl.ds(3 - my, 1), Bt],
                    rbuf.at[pl.ds(5, 1)], xp)
        rs2a.start()
        rs2b.start()
        rs2a.wait()
        rs2b.wait()
        partbuf[pl.ds(my, 1), A] = (
            partbuf[pl.ds(my, 1), A] + rbuf[pl.ds(4, 1)]
        )
        partbuf[pl.ds(my, 1), Bt] = (
            partbuf[pl.ds(my, 1), Bt] + rbuf[pl.ds(5, 1)]
        )

        ag1a = copy(5, partbuf.at[pl.ds(my, 1), A],
                    partbuf.at[pl.ds(my, 1), A], yp)
        ag1b = copy(6, partbuf.at[pl.ds(my, 1), Bt],
                    partbuf.at[pl.ds(my, 1), Bt], xp)
        ag2a_own = copy(7, partbuf.at[pl.ds(my, 1), A],
                        partbuf.at[pl.ds(my, 1), A], xp)
        ag2b_own = copy(8, partbuf.at[pl.ds(my, 1), Bt],
                        partbuf.at[pl.ds(my, 1), Bt], yp)
        ag1a.start()
        ag1b.start()
        ag2a_own.start()
        ag2b_own.start()

        ag1a.wait()
        ag2a_rel = copy(9, partbuf.at[pl.ds(yp, 1), A],
                        partbuf.at[pl.ds(yp, 1), A], xp)
        ag2a_rel.start()
        ag1b.wait()
        ag2b_rel = copy(10, partbuf.at[pl.ds(3 - my, 1), Bt],
                        partbuf.at[pl.ds(3 - my, 1), Bt], yp)
        ag2b_rel.start()

        ag2a_own.wait()
        ag2b_own.wait()
        ag2a_rel.wait()
        ag2b_rel.wait()

        out_ref[...] = partbuf[...].astype(jnp.float32)

    return pl.pallas_call(
        body,
        out_shape=jax.ShapeDtypeStruct((B, S, N), jnp.float32),
        in_specs=[
            pl.BlockSpec(memory_space=pltpu.VMEM),
            pl.BlockSpec(memory_space=pltpu.VMEM),
            pl.BlockSpec(memory_space=pltpu.VMEM),
        ],
        out_specs=pl.BlockSpec(memory_space=pltpu.VMEM),
        scratch_shapes=[
            pltpu.VMEM((B, S, N), jnp.bfloat16),
            pltpu.VMEM((6, H, N), jnp.bfloat16),
            pltpu.SemaphoreType.DMA((N_SEM,)),
            pltpu.SemaphoreType.DMA((N_SEM,)),
        ],
        compiler_params=pltpu.CompilerParams(collective_id=0),
    )(x, k, Wp)


# device time: 52638 ns/iter; 3.0249x vs baseline; 1.0434x over previous
import jax
import jax.numpy as jnp
from jax import lax
from jax.experimental import pallas as pl
from jax.experimental.pallas import tpu as pltpu

N_DEV = 4
N_SEM = 11


def kernel(x, k, Wp):
    B, S, C = x.shape
    T = k.shape[0]
    N = Wp.shape[1]
    H = S // 2

    def body(x_ref, k_ref, w_ref, out_ref, partbuf, rbuf,
             send_sems, recv_sems):
        my = lax.axis_index("i")
        r = lax.rem(my, 2)
        xp = 3 - my
        yp = my + 1 - 2 * r
        kps = my - r
        sps = 2 - kps
        srow2 = 2 - my + 2 * r

        barrier = pltpu.get_barrier_semaphore()
        for nbr in (xp, yp):
            pl.semaphore_signal(
                barrier, inc=1,
                device_id=(nbr,), device_id_type=pl.DeviceIdType.MESH,
            )
        pl.semaphore_wait(barrier, 2)

        kb = k_ref[...].astype(jnp.bfloat16)
        w = w_ref[...].astype(jnp.bfloat16)

        def compute_sub(q, u):
            halo = T - 1
            if u == 0:
                xq = x_ref[pl.ds(q, 1), pl.ds(0, H)].astype(jnp.bfloat16)
                xpad = jnp.concatenate(
                    [jnp.zeros((1, halo, C), jnp.bfloat16), xq], axis=1
                )
            else:
                xpad = x_ref[pl.ds(q, 1), pl.ds(u * H - halo, H + halo)]
                xpad = xpad.astype(jnp.bfloat16)
            y = xpad[:, halo:halo + H, :] * kb[T - 1]
            for t in range(T - 1):
                y = y + xpad[:, t:t + H, :] * kb[t]
            a = y * jax.nn.sigmoid(y)
            return jnp.dot(
                a.reshape(H, C), w, preferred_element_type=jnp.float32
            ).reshape(1, H, N).astype(jnp.bfloat16)

        def put_sub(q, u):
            partbuf[pl.ds(q, 1), pl.ds(u * H, H)] = compute_sub(q, u)

        def copy(i, src, dst, dev):
            return pltpu.make_async_remote_copy(
                src_ref=src, dst_ref=dst,
                send_sem=send_sems.at[i], recv_sem=recv_sems.at[i],
                device_id=(dev,), device_id_type=pl.DeviceIdType.MESH,
            )

        A = pl.ds(0, H)
        Bt = pl.ds(H, H)

        put_sub(srow2, 0)
        put_sub(3 - my, 0)
        rs1a = copy(0, partbuf.at[pl.ds(sps, 2), A],
                    rbuf.at[pl.ds(0, 2)], xp)
        rs1a.start()
        put_sub(srow2, 1)
        rs1b2 = copy(2, partbuf.at[pl.ds(srow2, 1), Bt],
                     rbuf.at[pl.ds(3, 1)], yp)
        rs1b2.start()
        put_sub(yp, 1)
        rs1b1 = copy(1, partbuf.at[pl.ds(yp, 1), Bt],
                     rbuf.at[pl.ds(2, 1)], yp)
        rs1b1.start()
        put_sub(yp, 0)
        put_sub(my, 0)
        put_sub(3 - my, 1)
        put_sub(my, 1)

        rs1b2.wait()
        partbuf[pl.ds(3 - my, 1), Bt] = (
            partbuf[pl.ds(3 - my, 1), Bt] + rbuf[pl.ds(3, 1)]
        )
        rs2b = copy(4, partbuf.at[pl.ds(3 - my, 1), Bt],
                    rbuf.at[pl.ds(5, 1)], xp)
        rs2b.start()
        rs1a.wait()
        partbuf[pl.ds(kps, 2), A] = (
            partbuf[pl.ds(kps, 2), A] + rbuf[pl.ds(0, 2)]
        )
        rs2a = copy(3, partbuf.at[pl.ds(yp, 1), A],
                    rbuf.at[pl.ds(4, 1)], yp)
        rs2a.start()
        rs1b1.wait()
        partbuf[pl.ds(my, 1), Bt] = (
            partbuf[pl.ds(my, 1), Bt] + rbuf[pl.ds(2, 1)]
        )

        rs2b.wait()
        partbuf[pl.ds(my, 1), Bt] = (
            partbuf[pl.ds(my, 1), Bt] + rbuf[pl.ds(5, 1)]
        )
        ag1b = copy(6, partbuf.at[pl.ds(my, 1), Bt],
                    partbuf.at[pl.ds(my, 1), Bt], xp)
        ag2b_own = copy(8, partbuf.at[pl.ds(my, 1), Bt],
                        partbuf.at[pl.ds(my, 1), Bt], yp)
        ag1b.start()
        ag2b_own.start()
        rs2a.wait()
        partbuf[pl.ds(my, 1), A] = (
            partbuf[pl.ds(my, 1), A] + rbuf[pl.ds(4, 1)]
        )
        ag1a = copy(5, partbuf.at[pl.ds(my, 1), A],
                    partbuf.at[pl.ds(my, 1), A], yp)
        ag2a_own = copy(7, partbuf.at[pl.ds(my, 1), A],
                        partbuf.at[pl.ds(my, 1), A], xp)
        ag1a.start()
        ag2a_own.start()

        out_ref[pl.ds(my, 1)] = partbuf[pl.ds(my, 1)].astype(jnp.float32)

        ag1a.wait()
        ag2a_rel = copy(9, partbuf.at[pl.ds(yp, 1), A],
                        partbuf.at[pl.ds(yp, 1), A], xp)
        ag2a_rel.start()
        out_ref[pl.ds(yp, 1), A] = partbuf[pl.ds(yp, 1), A].astype(jnp.float32)
        ag1b.wait()
        ag2b_rel = copy(10, partbuf.at[pl.ds(3 - my, 1), Bt],
                        partbuf.at[pl.ds(3 - my, 1), Bt], yp)
        ag2b_rel.start()
        out_ref[pl.ds(3 - my, 1), Bt] = (
            partbuf[pl.ds(3 - my, 1), Bt].astype(jnp.float32)
        )

        ag2a_own.wait()
        out_ref[pl.ds(3 - my, 1), A] = (
            partbuf[pl.ds(3 - my, 1), A].astype(jnp.float32)
        )
        ag2b_own.wait()
        out_ref[pl.ds(yp, 1), Bt] = (
            partbuf[pl.ds(yp, 1), Bt].astype(jnp.float32)
        )
        ag2a_rel.wait()
        out_ref[pl.ds(srow2, 1), A] = (
            partbuf[pl.ds(srow2, 1), A].astype(jnp.float32)
        )
        ag2b_rel.wait()
        out_ref[pl.ds(srow2, 1), Bt] = (
            partbuf[pl.ds(srow2, 1), Bt].astype(jnp.float32)
        )

    return pl.pallas_call(
        body,
        out_shape=jax.ShapeDtypeStruct((B, S, N), jnp.float32),
        in_specs=[
            pl.BlockSpec(memory_space=pltpu.VMEM),
            pl.BlockSpec(memory_space=pltpu.VMEM),
            pl.BlockSpec(memory_space=pltpu.VMEM),
        ],
        out_specs=pl.BlockSpec(memory_space=pltpu.VMEM),
        scratch_shapes=[
            pltpu.VMEM((B, S, N), jnp.bfloat16),
            pltpu.VMEM((6, H, N), jnp.bfloat16),
            pltpu.SemaphoreType.DMA((N_SEM,)),
            pltpu.SemaphoreType.DMA((N_SEM,)),
        ],
        compiler_params=pltpu.CompilerParams(collective_id=0),
    )(x, k, Wp)


# device time: 51813 ns/iter; 3.0730x vs baseline; 1.0159x over previous
import jax
import jax.numpy as jnp
from jax import lax
from jax.experimental import pallas as pl
from jax.experimental.pallas import tpu as pltpu

N_DEV = 4
N_SEM = 12


def kernel(x, k, Wp):
    B, S, C = x.shape
    T = k.shape[0]
    N = Wp.shape[1]
    H = S // 2

    def body(x_ref, k_ref, w_ref, out_ref, partbuf, rbuf,
             send_sems, recv_sems):
        my = lax.axis_index("i")
        r = lax.rem(my, 2)
        xp = 3 - my
        yp = my + 1 - 2 * r
        kps = my - r
        sps = 2 - kps
        srow2 = 2 - my + 2 * r

        barrier = pltpu.get_barrier_semaphore()
        for nbr in (xp, yp):
            pl.semaphore_signal(
                barrier, inc=1,
                device_id=(nbr,), device_id_type=pl.DeviceIdType.MESH,
            )
        pl.semaphore_wait(barrier, 2)

        kb = k_ref[...].astype(jnp.bfloat16)
        w = w_ref[...].astype(jnp.bfloat16)

        def compute_sub(q, u):
            halo = T - 1
            if u == 0:
                xq = x_ref[pl.ds(q, 1), pl.ds(0, H)].astype(jnp.bfloat16)
                xpad = jnp.concatenate(
                    [jnp.zeros((1, halo, C), jnp.bfloat16), xq], axis=1
                )
            else:
                xpad = x_ref[pl.ds(q, 1), pl.ds(u * H - halo, H + halo)]
                xpad = xpad.astype(jnp.bfloat16)
            y = xpad[:, halo:halo + H, :] * kb[T - 1]
            for t in range(T - 1):
                y = y + xpad[:, t:t + H, :] * kb[t]
            a = y * jax.nn.sigmoid(y)
            return jnp.dot(
                a.reshape(H, C), w, preferred_element_type=jnp.float32
            ).reshape(1, H, N).astype(jnp.bfloat16)

        def put_sub(q, u):
            partbuf[pl.ds(q, 1), pl.ds(u * H, H)] = compute_sub(q, u)

        def copy(i, src, dst, dev):
            return pltpu.make_async_remote_copy(
                src_ref=src, dst_ref=dst,
                send_sem=send_sems.at[i], recv_sem=recv_sems.at[i],
                device_id=(dev,), device_id_type=pl.DeviceIdType.MESH,
            )

        A = pl.ds(0, H)
        Bt = pl.ds(H, H)

        put_sub(srow2, 0)
        rs1a0 = copy(0, partbuf.at[pl.ds(srow2, 1), A],
                     rbuf.at[pl.ds(0, 1)], xp)
        rs1a0.start()
        put_sub(srow2, 1)
        rs1b2 = copy(2, partbuf.at[pl.ds(srow2, 1), Bt],
                     rbuf.at[pl.ds(3, 1)], yp)
        rs1b2.start()
        put_sub(3 - my, 0)
        rs1a1 = copy(11, partbuf.at[pl.ds(3 - my, 1), A],
                     rbuf.at[pl.ds(1, 1)], xp)
        rs1a1.start()
        put_sub(yp, 1)
        rs1b1 = copy(1, partbuf.at[pl.ds(yp, 1), Bt],
                     rbuf.at[pl.ds(2, 1)], yp)
        rs1b1.start()
        put_sub(yp, 0)
        put_sub(my, 0)
        put_sub(3 - my, 1)
        put_sub(my, 1)

        rs1b2.wait()
        partbuf[pl.ds(3 - my, 1), Bt] = (
            partbuf[pl.ds(3 - my, 1), Bt] + rbuf[pl.ds(3, 1)]
        )
        rs2b = copy(4, partbuf.at[pl.ds(3 - my, 1), Bt],
                    rbuf.at[pl.ds(5, 1)], xp)
        rs2b.start()
        rs1a0.wait()
        partbuf[pl.ds(yp, 1), A] = (
            partbuf[pl.ds(yp, 1), A] + rbuf[pl.ds(0, 1)]
        )
        rs2a = copy(3, partbuf.at[pl.ds(yp, 1), A],
                    rbuf.at[pl.ds(4, 1)], yp)
        rs2a.start()
        rs1a1.wait()
        partbuf[pl.ds(my, 1), A] = (
            partbuf[pl.ds(my, 1), A] + rbuf[pl.ds(1, 1)]
        )
        rs1b1.wait()
        partbuf[pl.ds(my, 1), Bt] = (
            partbuf[pl.ds(my, 1), Bt] + rbuf[pl.ds(2, 1)]
        )

        rs2b.wait()
        partbuf[pl.ds(my, 1), Bt] = (
            partbuf[pl.ds(my, 1), Bt] + rbuf[pl.ds(5, 1)]
        )
        ag1b = copy(6, partbuf.at[pl.ds(my, 1), Bt],
                    partbuf.at[pl.ds(my, 1), Bt], xp)
        ag2b_own = copy(8, partbuf.at[pl.ds(my, 1), Bt],
                        partbuf.at[pl.ds(my, 1), Bt], yp)
        ag1b.start()
        ag2b_own.start()
        rs2a.wait()
        partbuf[pl.ds(my, 1), A] = (
            partbuf[pl.ds(my, 1), A] + rbuf[pl.ds(4, 1)]
        )
        ag1a = copy(5, partbuf.at[pl.ds(my, 1), A],
                    partbuf.at[pl.ds(my, 1), A], yp)
        ag2a_own = copy(7, partbuf.at[pl.ds(my, 1), A],
                        partbuf.at[pl.ds(my, 1), A], xp)
        ag1a.start()
        ag2a_own.start()

        out_ref[pl.ds(my, 1)] = partbuf[pl.ds(my, 1)].astype(jnp.float32)

        ag1a.wait()
        ag2a_rel = copy(9, partbuf.at[pl.ds(yp, 1), A],
                        partbuf.at[pl.ds(yp, 1), A], xp)
        ag2a_rel.start()
        out_ref[pl.ds(yp, 1), A] = partbuf[pl.ds(yp, 1), A].astype(jnp.float32)
        ag1b.wait()
        ag2b_rel = copy(10, partbuf.at[pl.ds(3 - my, 1), Bt],
                        partbuf.at[pl.ds(3 - my, 1), Bt], yp)
        ag2b_rel.start()
        out_ref[pl.ds(3 - my, 1), Bt] = (
            partbuf[pl.ds(3 - my, 1), Bt].astype(jnp.float32)
        )

        ag2a_own.wait()
        out_ref[pl.ds(3 - my, 1), A] = (
            partbuf[pl.ds(3 - my, 1), A].astype(jnp.float32)
        )
        ag2b_own.wait()
        out_ref[pl.ds(yp, 1), Bt] = (
            partbuf[pl.ds(yp, 1), Bt].astype(jnp.float32)
        )
        ag2a_rel.wait()
        out_ref[pl.ds(srow2, 1), A] = (
            partbuf[pl.ds(srow2, 1), A].astype(jnp.float32)
        )
        ag2b_rel.wait()
        out_ref[pl.ds(srow2, 1), Bt] = (
            partbuf[pl.ds(srow2, 1), Bt].astype(jnp.float32)
        )

    return pl.pallas_call(
        body,
        out_shape=jax.ShapeDtypeStruct((B, S, N), jnp.float32),
        in_specs=[
            pl.BlockSpec(memory_space=pltpu.VMEM),
            pl.BlockSpec(memory_space=pltpu.VMEM),
            pl.BlockSpec(memory_space=pltpu.VMEM),
        ],
        out_specs=pl.BlockSpec(memory_space=pltpu.VMEM),
        scratch_shapes=[
            pltpu.VMEM((B, S, N), jnp.bfloat16),
            pltpu.VMEM((6, H, N), jnp.bfloat16),
            pltpu.SemaphoreType.DMA((N_SEM,)),
            pltpu.SemaphoreType.DMA((N_SEM,)),
        ],
        compiler_params=pltpu.CompilerParams(collective_id=0),
    )(x, k, Wp)


# device time: 50497 ns/iter; 3.1531x vs baseline; 1.0261x over previous
import jax
import jax.numpy as jnp
from jax import lax
from jax.experimental import pallas as pl
from jax.experimental.pallas import tpu as pltpu

N_DEV = 4
N_SEM = 12


def kernel(x, k, Wp):
    B, S, C = x.shape
    T = k.shape[0]
    N = Wp.shape[1]
    H = S // 2

    def body(x_ref, k_ref, w_ref, out_ref, partbuf, rbuf,
             send_sems, recv_sems):
        my = lax.axis_index("i")
        r = lax.rem(my, 2)
        xp = 3 - my
        yp = my + 1 - 2 * r
        kps = my - r
        sps = 2 - kps
        srow2 = 2 - my + 2 * r

        barrier = pltpu.get_barrier_semaphore()
        for nbr in (xp, yp):
            pl.semaphore_signal(
                barrier, inc=1,
                device_id=(nbr,), device_id_type=pl.DeviceIdType.MESH,
            )
        pl.semaphore_wait(barrier, 2)

        kb = k_ref[...].astype(jnp.bfloat16)
        w = w_ref[...].astype(jnp.bfloat16)

        def compute_sub(q, u):
            halo = T - 1
            if u == 0:
                xq = x_ref[pl.ds(q, 1), pl.ds(0, H)].astype(jnp.bfloat16)
                xpad = jnp.concatenate(
                    [jnp.zeros((1, halo, C), jnp.bfloat16), xq], axis=1
                )
            else:
                xpad = x_ref[pl.ds(q, 1), pl.ds(u * H - halo, H + halo)]
                xpad = xpad.astype(jnp.bfloat16)
            y = xpad[:, halo:halo + H, :] * kb[T - 1]
            for t in range(T - 1):
                y = y + xpad[:, t:t + H, :] * kb[t]
            a = y * jax.nn.sigmoid(y)
            return jnp.dot(
                a.reshape(H, C), w, preferred_element_type=jnp.float32
            ).reshape(1, H, N).astype(jnp.bfloat16)

        def put_sub(q, u):
            partbuf[pl.ds(q, 1), pl.ds(u * H, H)] = compute_sub(q, u)

        def copy(i, src, dst, dev):
            return pltpu.make_async_remote_copy(
                src_ref=src, dst_ref=dst,
                send_sem=send_sems.at[i], recv_sem=recv_sems.at[i],
                device_id=(dev,), device_id_type=pl.DeviceIdType.MESH,
            )

        A = pl.ds(0, H)
        Bt = pl.ds(H, H)

        put_sub(srow2, 0)
        rs1a0 = copy(0, partbuf.at[pl.ds(srow2, 1), A],
                     rbuf.at[pl.ds(0, 1)], xp)
        rs1a0.start()
        put_sub(srow2, 1)
        rs1b2 = copy(2, partbuf.at[pl.ds(srow2, 1), Bt],
                     rbuf.at[pl.ds(3, 1)], yp)
        rs1b2.start()
        put_sub(3 - my, 0)
        rs1a1 = copy(11, partbuf.at[pl.ds(3 - my, 1), A],
                     rbuf.at[pl.ds(1, 1)], xp)
        rs1a1.start()
        put_sub(yp, 1)
        rs1b1 = copy(1, partbuf.at[pl.ds(yp, 1), Bt],
                     rbuf.at[pl.ds(2, 1)], yp)
        rs1b1.start()
        put_sub(yp, 0)
        put_sub(my, 0)
        put_sub(3 - my, 1)
        put_sub(my, 1)

        rs1b2.wait()
        partbuf[pl.ds(3 - my, 1), Bt] = (
            partbuf[pl.ds(3 - my, 1), Bt] + rbuf[pl.ds(3, 1)]
        )
        rs2b = copy(4, partbuf.at[pl.ds(3 - my, 1), Bt],
                    rbuf.at[pl.ds(5, 1)], xp)
        rs2b.start()
        rs1a0.wait()
        partbuf[pl.ds(yp, 1), A] = (
            partbuf[pl.ds(yp, 1), A] + rbuf[pl.ds(0, 1)]
        )
        rs2a = copy(3, partbuf.at[pl.ds(yp, 1), A],
                    rbuf.at[pl.ds(4, 1)], yp)
        rs2a.start()
        rs1a1.wait()
        partbuf[pl.ds(my, 1), A] = (
            partbuf[pl.ds(my, 1), A] + rbuf[pl.ds(1, 1)]
        )
        rs1b1.wait()
        partbuf[pl.ds(my, 1), Bt] = (
            partbuf[pl.ds(my, 1), Bt] + rbuf[pl.ds(2, 1)]
        )

        rs2b.wait()
        partbuf[pl.ds(my, 1), Bt] = (
            partbuf[pl.ds(my, 1), Bt] + rbuf[pl.ds(5, 1)]
        )
        ag1b = copy(6, partbuf.at[pl.ds(my, 1), Bt],
                    out_ref.at[pl.ds(my, 1), Bt], xp)
        ag2b_own = copy(8, partbuf.at[pl.ds(my, 1), Bt],
                        out_ref.at[pl.ds(my, 1), Bt], yp)
        ag1b.start()
        ag2b_own.start()
        rs2a.wait()
        partbuf[pl.ds(my, 1), A] = (
            partbuf[pl.ds(my, 1), A] + rbuf[pl.ds(4, 1)]
        )
        ag1a = copy(5, partbuf.at[pl.ds(my, 1), A],
                    out_ref.at[pl.ds(my, 1), A], yp)
        ag2a_own = copy(7, partbuf.at[pl.ds(my, 1), A],
                        out_ref.at[pl.ds(my, 1), A], xp)
        ag1a.start()
        ag2a_own.start()

        out_ref[pl.ds(my, 1)] = partbuf[pl.ds(my, 1)]

        ag1a.wait()
        ag2a_rel = copy(9, out_ref.at[pl.ds(yp, 1), A],
                        out_ref.at[pl.ds(yp, 1), A], xp)
        ag2a_rel.start()
        ag1b.wait()
        ag2b_rel = copy(10, out_ref.at[pl.ds(3 - my, 1), Bt],
                        out_ref.at[pl.ds(3 - my, 1), Bt], yp)
        ag2b_rel.start()

        ag2a_own.wait()
        ag2b_own.wait()
        ag2a_rel.wait()
        ag2b_rel.wait()

    return pl.pallas_call(
        body,
        out_shape=jax.ShapeDtypeStruct((B, S, N), jnp.bfloat16),
        in_specs=[
            pl.BlockSpec(memory_space=pltpu.VMEM),
            pl.BlockSpec(memory_space=pltpu.VMEM),
            pl.BlockSpec(memory_space=pltpu.VMEM),
        ],
        out_specs=pl.BlockSpec(memory_space=pltpu.VMEM),
        scratch_shapes=[
            pltpu.VMEM((B, S, N), jnp.bfloat16),
            pltpu.VMEM((6, H, N), jnp.bfloat16),
            pltpu.SemaphoreType.DMA((N_SEM,)),
            pltpu.SemaphoreType.DMA((N_SEM,)),
        ],
        compiler_params=pltpu.CompilerParams(collective_id=0),
    )(x, k, Wp)


# device time: 48484 ns/iter; 3.2840x vs baseline; 1.0415x over previous
import jax
import jax.numpy as jnp
from jax import lax
from jax.experimental import pallas as pl
from jax.experimental.pallas import tpu as pltpu

N_DEV = 4
N_SEM = 16


def kernel(x, k, Wp):
    B, S, C = x.shape
    T = k.shape[0]
    N = Wp.shape[1]
    H = S // 2

    def body(x_ref, k_ref, w_ref, out_ref, partbuf, rbuf,
             send_sems, recv_sems):
        my = lax.axis_index("i")
        r = lax.rem(my, 2)
        xp = 3 - my
        yp = my + 1 - 2 * r
        kps = my - r
        sps = 2 - kps
        srow2 = 2 - my + 2 * r

        barrier = pltpu.get_barrier_semaphore()
        for nbr in (xp, yp):
            pl.semaphore_signal(
                barrier, inc=1,
                device_id=(nbr,), device_id_type=pl.DeviceIdType.MESH,
            )
        pl.semaphore_wait(barrier, 2)

        kb = k_ref[...].astype(jnp.bfloat16)
        w = w_ref[...].astype(jnp.bfloat16)

        def compute_sub(q, u):
            halo = T - 1
            if u == 0:
                xq = x_ref[pl.ds(q, 1), pl.ds(0, H)].astype(jnp.bfloat16)
                xpad = jnp.concatenate(
                    [jnp.zeros((1, halo, C), jnp.bfloat16), xq], axis=1
                )
            else:
                xpad = x_ref[pl.ds(q, 1), pl.ds(u * H - halo, H + halo)]
                xpad = xpad.astype(jnp.bfloat16)
            y = xpad[:, halo:halo + H, :] * kb[T - 1]
            for t in range(T - 1):
                y = y + xpad[:, t:t + H, :] * kb[t]
            a = y * jax.nn.sigmoid(y)
            return jnp.dot(
                a.reshape(H, C), w, preferred_element_type=jnp.float32
            ).reshape(1, H, N).astype(jnp.bfloat16)

        def put_sub(q, u):
            partbuf[pl.ds(q, 1), pl.ds(u * H, H)] = compute_sub(q, u)

        def copy(i, src, dst, dev):
            return pltpu.make_async_remote_copy(
                src_ref=src, dst_ref=dst,
                send_sem=send_sems.at[i], recv_sem=recv_sems.at[i],
                device_id=(dev,), device_id_type=pl.DeviceIdType.MESH,
            )

        A = pl.ds(0, H)
        Bt = pl.ds(H, H)
        H2 = H // 2
        A0, A1 = pl.ds(0, H2), pl.ds(H2, H2)
        B0, B1 = pl.ds(H, H2), pl.ds(H + H2, H2)

        put_sub(srow2, 0)
        rs1a0 = copy(0, partbuf.at[pl.ds(srow2, 1), A],
                     rbuf.at[pl.ds(0, 1)], xp)
        rs1a0.start()
        put_sub(srow2, 1)
        rs1b2 = copy(2, partbuf.at[pl.ds(srow2, 1), Bt],
                     rbuf.at[pl.ds(3, 1)], yp)
        rs1b2.start()
        put_sub(3 - my, 0)
        rs1a1 = copy(11, partbuf.at[pl.ds(3 - my, 1), A],
                     rbuf.at[pl.ds(1, 1)], xp)
        rs1a1.start()
        put_sub(yp, 1)
        rs1b1 = copy(1, partbuf.at[pl.ds(yp, 1), Bt],
                     rbuf.at[pl.ds(2, 1)], yp)
        rs1b1.start()
        put_sub(yp, 0)
        put_sub(my, 0)
        put_sub(3 - my, 1)
        put_sub(my, 1)

        rs1b2.wait()
        partbuf[pl.ds(3 - my, 1), Bt] = (
            partbuf[pl.ds(3 - my, 1), Bt] + rbuf[pl.ds(3, 1)]
        )
        rs2b = copy(4, partbuf.at[pl.ds(3 - my, 1), Bt],
                    rbuf.at[pl.ds(5, 1)], xp)
        rs2b.start()
        rs1a0.wait()
        partbuf[pl.ds(yp, 1), A] = (
            partbuf[pl.ds(yp, 1), A] + rbuf[pl.ds(0, 1)]
        )
        rs2a = copy(3, partbuf.at[pl.ds(yp, 1), A],
                    rbuf.at[pl.ds(4, 1)], yp)
        rs2a.start()
        rs1a1.wait()
        partbuf[pl.ds(my, 1), A] = (
            partbuf[pl.ds(my, 1), A] + rbuf[pl.ds(1, 1)]
        )
        rs1b1.wait()
        partbuf[pl.ds(my, 1), Bt] = (
            partbuf[pl.ds(my, 1), Bt] + rbuf[pl.ds(2, 1)]
        )

        rs2b.wait()
        partbuf[pl.ds(my, 1), Bt] = (
            partbuf[pl.ds(my, 1), Bt] + rbuf[pl.ds(5, 1)]
        )
        ag1b0 = copy(6, partbuf.at[pl.ds(my, 1), B0],
                     out_ref.at[pl.ds(my, 1), B0], xp)
        ag1b1 = copy(14, partbuf.at[pl.ds(my, 1), B1],
                     out_ref.at[pl.ds(my, 1), B1], xp)
        ag2b_own = copy(8, partbuf.at[pl.ds(my, 1), Bt],
                        out_ref.at[pl.ds(my, 1), Bt], yp)
        ag1b0.start()
        ag1b1.start()
        ag2b_own.start()
        rs2a.wait()
        partbuf[pl.ds(my, 1), A] = (
            partbuf[pl.ds(my, 1), A] + rbuf[pl.ds(4, 1)]
        )
        ag1a0 = copy(5, partbuf.at[pl.ds(my, 1), A0],
                     out_ref.at[pl.ds(my, 1), A0], yp)
        ag1a1 = copy(12, partbuf.at[pl.ds(my, 1), A1],
                     out_ref.at[pl.ds(my, 1), A1], yp)
        ag2a_own = copy(7, partbuf.at[pl.ds(my, 1), A],
                        out_ref.at[pl.ds(my, 1), A], xp)
        ag1a0.start()
        ag1a1.start()
        ag2a_own.start()

        out_ref[pl.ds(my, 1)] = partbuf[pl.ds(my, 1)]

        ag1a0.wait()
        ag2a_rel0 = copy(9, out_ref.at[pl.ds(yp, 1), A0],
                         out_ref.at[pl.ds(yp, 1), A0], xp)
        ag2a_rel0.start()
        ag1b0.wait()
        ag2b_rel0 = copy(10, out_ref.at[pl.ds(3 - my, 1), B0],
                         out_ref.at[pl.ds(3 - my, 1), B0], yp)
        ag2b_rel0.start()
        ag1a1.wait()
        ag2a_rel1 = copy(13, out_ref.at[pl.ds(yp, 1), A1],
                         out_ref.at[pl.ds(yp, 1), A1], xp)
        ag2a_rel1.start()
        ag1b1.wait()
        ag2b_rel1 = copy(15, out_ref.at[pl.ds(3 - my, 1), B1],
                         out_ref.at[pl.ds(3 - my, 1), B1], yp)
        ag2b_rel1.start()

        ag2a_own.wait()
        ag2b_own.wait()
        ag2a_rel0.wait()
        ag2a_rel1.wait()
        ag2b_rel0.wait()
        ag2b_rel1.wait()

    return pl.pallas_call(
        body,
        out_shape=jax.ShapeDtypeStruct((B, S, N), jnp.bfloat16),
        in_specs=[
            pl.BlockSpec(memory_space=pltpu.VMEM),
            pl.BlockSpec(memory_space=pltpu.VMEM),
            pl.BlockSpec(memory_space=pltpu.VMEM),
        ],
        out_specs=pl.BlockSpec(memory_space=pltpu.VMEM),
        scratch_shapes=[
            pltpu.VMEM((B, S, N), jnp.bfloat16),
            pltpu.VMEM((6, H, N), jnp.bfloat16),
            pltpu.SemaphoreType.DMA((N_SEM,)),
            pltpu.SemaphoreType.DMA((N_SEM,)),
        ],
        compiler_params=pltpu.CompilerParams(collective_id=0),
    )(x, k, Wp)


# device time: 48449 ns/iter; 3.2864x vs baseline; 1.0007x over previous
import jax
import jax.numpy as jnp
from jax import lax
from jax.experimental import pallas as pl
from jax.experimental.pallas import tpu as pltpu

N_DEV = 4
N_SEM = 18


def kernel(x, k, Wp):
    B, S, C = x.shape
    T = k.shape[0]
    N = Wp.shape[1]
    H = S // 2

    def body(x_ref, k_ref, w_ref, out_ref, partbuf, rbuf,
             send_sems, recv_sems):
        my = lax.axis_index("i")
        r = lax.rem(my, 2)
        xp = 3 - my
        yp = my + 1 - 2 * r
        kps = my - r
        sps = 2 - kps
        srow2 = 2 - my + 2 * r

        barrier = pltpu.get_barrier_semaphore()
        for nbr in (xp, yp):
            pl.semaphore_signal(
                barrier, inc=1,
                device_id=(nbr,), device_id_type=pl.DeviceIdType.MESH,
            )
        pl.semaphore_wait(barrier, 2)

        kb = k_ref[...].astype(jnp.bfloat16)
        w = w_ref[...].astype(jnp.bfloat16)

        def compute_sub(q, u):
            halo = T - 1
            if u == 0:
                xq = x_ref[pl.ds(q, 1), pl.ds(0, H)].astype(jnp.bfloat16)
                xpad = jnp.concatenate(
                    [jnp.zeros((1, halo, C), jnp.bfloat16), xq], axis=1
                )
            else:
                xpad = x_ref[pl.ds(q, 1), pl.ds(u * H - halo, H + halo)]
                xpad = xpad.astype(jnp.bfloat16)
            y = xpad[:, halo:halo + H, :] * kb[T - 1]
            for t in range(T - 1):
                y = y + xpad[:, t:t + H, :] * kb[t]
            a = y * jax.nn.sigmoid(y)
            return jnp.dot(
                a.reshape(H, C), w, preferred_element_type=jnp.float32
            ).reshape(1, H, N).astype(jnp.bfloat16)

        def put_sub(q, u):
            partbuf[pl.ds(q, 1), pl.ds(u * H, H)] = compute_sub(q, u)

        def copy(i, src, dst, dev):
            return pltpu.make_async_remote_copy(
                src_ref=src, dst_ref=dst,
                send_sem=send_sems.at[i], recv_sem=recv_sems.at[i],
                device_id=(dev,), device_id_type=pl.DeviceIdType.MESH,
            )

        A = pl.ds(0, H)
        Bt = pl.ds(H, H)
        H2 = H // 2
        A0, A1 = pl.ds(0, H2), pl.ds(H2, H2)
        B0, B1 = pl.ds(H, H2), pl.ds(H + H2, H2)

        put_sub(srow2, 0)
        rs1a0 = copy(0, partbuf.at[pl.ds(srow2, 1), A],
                     rbuf.at[pl.ds(0, 1)], xp)
        rs1a0.start()
        put_sub(srow2, 1)
        rs1b2 = copy(2, partbuf.at[pl.ds(srow2, 1), Bt],
                     rbuf.at[pl.ds(3, 1)], yp)
        rs1b2.start()
        put_sub(3 - my, 0)
        rs1a1 = copy(11, partbuf.at[pl.ds(3 - my, 1), A],
                     rbuf.at[pl.ds(1, 1)], xp)
        rs1a1.start()
        put_sub(yp, 1)
        rs1b1 = copy(1, partbuf.at[pl.ds(yp, 1), Bt],
                     rbuf.at[pl.ds(2, 1)], yp)
        rs1b1.start()
        put_sub(yp, 0)
        put_sub(my, 0)
        put_sub(3 - my, 1)
        put_sub(my, 1)

        rs1b2.wait()
        partbuf[pl.ds(3 - my, 1), Bt] = (
            partbuf[pl.ds(3 - my, 1), Bt] + rbuf[pl.ds(3, 1)]
        )
        rs2b0 = copy(4, partbuf.at[pl.ds(3 - my, 1), B0],
                     rbuf.at[pl.ds(5, 1), pl.ds(0, H2)], xp)
        rs2b1 = copy(17, partbuf.at[pl.ds(3 - my, 1), B1],
                     rbuf.at[pl.ds(5, 1), pl.ds(H2, H2)], xp)
        rs2b0.start()
        rs2b1.start()
        rs1a0.wait()
        partbuf[pl.ds(yp, 1), A] = (
            partbuf[pl.ds(yp, 1), A] + rbuf[pl.ds(0, 1)]
        )
        rs2a0 = copy(3, partbuf.at[pl.ds(yp, 1), A0],
                     rbuf.at[pl.ds(4, 1), pl.ds(0, H2)], yp)
        rs2a1 = copy(16, partbuf.at[pl.ds(yp, 1), A1],
                     rbuf.at[pl.ds(4, 1), pl.ds(H2, H2)], yp)
        rs2a0.start()
        rs2a1.start()
        rs1a1.wait()
        partbuf[pl.ds(my, 1), A] = (
            partbuf[pl.ds(my, 1), A] + rbuf[pl.ds(1, 1)]
        )
        rs1b1.wait()
        partbuf[pl.ds(my, 1), Bt] = (
            partbuf[pl.ds(my, 1), Bt] + rbuf[pl.ds(2, 1)]
        )

        rs2b0.wait()
        partbuf[pl.ds(my, 1), B0] = (
            partbuf[pl.ds(my, 1), B0] + rbuf[pl.ds(5, 1), pl.ds(0, H2)]
        )
        ag1b0 = copy(6, partbuf.at[pl.ds(my, 1), B0],
                     out_ref.at[pl.ds(my, 1), B0], xp)
        ag1b0.start()
        rs2b1.wait()
        partbuf[pl.ds(my, 1), B1] = (
            partbuf[pl.ds(my, 1), B1] + rbuf[pl.ds(5, 1), pl.ds(H2, H2)]
        )
        ag1b1 = copy(14, partbuf.at[pl.ds(my, 1), B1],
                     out_ref.at[pl.ds(my, 1), B1], xp)
        ag2b_own = copy(8, partbuf.at[pl.ds(my, 1), Bt],
                        out_ref.at[pl.ds(my, 1), Bt], yp)
        ag1b1.start()
        ag2b_own.start()
        rs2a0.wait()
        partbuf[pl.ds(my, 1), A0] = (
            partbuf[pl.ds(my, 1), A0] + rbuf[pl.ds(4, 1), pl.ds(0, H2)]
        )
        ag1a0 = copy(5, partbuf.at[pl.ds(my, 1), A0],
                     out_ref.at[pl.ds(my, 1), A0], yp)
        ag1a0.start()
        rs2a1.wait()
        partbuf[pl.ds(my, 1), A1] = (
            partbuf[pl.ds(my, 1), A1] + rbuf[pl.ds(4, 1), pl.ds(H2, H2)]
        )
        ag1a1 = copy(12, partbuf.at[pl.ds(my, 1), A1],
                     out_ref.at[pl.ds(my, 1), A1], yp)
        ag2a_own = copy(7, partbuf.at[pl.ds(my, 1), A],
                        out_ref.at[pl.ds(my, 1), A], xp)
        ag1a1.start()
        ag2a_own.start()

        out_ref[pl.ds(my, 1)] = partbuf[pl.ds(my, 1)]

        ag1a0.wait()
        ag2a_rel0 = copy(9, out_ref.at[pl.ds(yp, 1), A0],
                         out_ref.at[pl.ds(yp, 1), A0], xp)
        ag2a_rel0.start()
        ag1b0.wait()
        ag2b_rel0 = copy(10, out_ref.at[pl.ds(3 - my, 1), B0],
                         out_ref.at[pl.ds(3 - my, 1), B0], yp)
        ag2b_rel0.start()
        ag1a1.wait()
        ag2a_rel1 = copy(13, out_ref.at[pl.ds(yp, 1), A1],
                         out_ref.at[pl.ds(yp, 1), A1], xp)
        ag2a_rel1.start()
        ag1b1.wait()
        ag2b_rel1 = copy(15, out_ref.at[pl.ds(3 - my, 1), B1],
                         out_ref.at[pl.ds(3 - my, 1), B1], yp)
        ag2b_rel1.start()

        ag2a_own.wait()
        ag2b_own.wait()
        ag2a_rel0.wait()
        ag2a_rel1.wait()
        ag2b_rel0.wait()
        ag2b_rel1.wait()

    return pl.pallas_call(
        body,
        out_shape=jax.ShapeDtypeStruct((B, S, N), jnp.bfloat16),
        in_specs=[
            pl.BlockSpec(memory_space=pltpu.VMEM),
            pl.BlockSpec(memory_space=pltpu.VMEM),
            pl.BlockSpec(memory_space=pltpu.VMEM),
        ],
        out_specs=pl.BlockSpec(memory_space=pltpu.VMEM),
        scratch_shapes=[
            pltpu.VMEM((B, S, N), jnp.bfloat16),
            pltpu.VMEM((6, H, N), jnp.bfloat16),
            pltpu.SemaphoreType.DMA((N_SEM,)),
            pltpu.SemaphoreType.DMA((N_SEM,)),
        ],
        compiler_params=pltpu.CompilerParams(collective_id=0),
    )(x, k, Wp)
